# baseline (speedup 1.0000x reference)
import functools

import jax
import jax.numpy as jnp
from jax import lax
from jax.experimental import pallas as pl
from jax.experimental.pallas import tpu as pltpu
from jax.experimental.pallas import tpu_sc as plsc

N = 1048576
TILE = 128
NT = N // TILE
NUM_CORES = 2
NUM_SUBCORES = 16
NW = NUM_CORES * NUM_SUBCORES
T_PER_W = NT // NW
T_CHUNK = 64
SC_CHUNKS = T_PER_W // T_CHUNK
LANES = 16

BT = 256
GRID = NT // BT


def _sc_body(ddiam_hbm, zero_v):
    wid = lax.axis_index("s") * NUM_CORES + lax.axis_index("c")
    zeros = jnp.zeros((LANES,), jnp.float32)

    @plsc.parallel_loop(0, T_CHUNK * (TILE // LANES), unroll=8)
    def zbody(g):
        zero_v[g // 8, pl.ds((g % 8) * LANES, LANES)] = zeros

    for c in range(SC_CHUNKS):
        t0 = wid * T_PER_W + c * T_CHUNK
        pltpu.sync_copy(zero_v, ddiam_hbm.at[pl.ds(t0, T_CHUNK)])


_sc_call = functools.partial(
    pl.kernel,
    out_type=jax.ShapeDtypeStruct((NT, TILE), jnp.float32),
    mesh=plsc.VectorSubcoreMesh(core_axis_name="c", subcore_axis_name="s"),
    compiler_params=pltpu.CompilerParams(
        needs_layout_passes=False, use_tc_tiling_on_sc=False),
    scratch_types=[pltpu.VMEM((T_CHUNK, TILE), jnp.float32)],
)(_sc_body)


def _tc_body(pos_ref, vel_ref, diam_ref, dpos_ref, dvel_ref):
    y = pos_ref[...].reshape(BT, 2, TILE)[:, 1, :]
    d = diam_ref[...]
    m = y <= 0.5 * d
    mv = jnp.broadcast_to(m[:, None, :], (BT, 2, TILE)).reshape(2 * BT, TILE)
    dpos_ref[...] = jnp.where(mv, 0.0, vel_ref[...])
    row = jax.lax.broadcasted_iota(jnp.int32, (2 * BT, TILE), 0)
    pat = jnp.where((row % 2) == 1, -20.0, 0.0)
    dvel_ref[...] = jnp.where(mv, 0.0, pat)


_tc_call = pl.pallas_call(
    _tc_body,
    grid=(GRID,),
    in_specs=[
        pl.BlockSpec((2 * BT, TILE), lambda i: (i, 0)),
        pl.BlockSpec((2 * BT, TILE), lambda i: (i, 0)),
        pl.BlockSpec((BT, TILE), lambda i: (i, 0)),
    ],
    out_specs=[
        pl.BlockSpec((2 * BT, TILE), lambda i: (i, 0)),
        pl.BlockSpec((2 * BT, TILE), lambda i: (i, 0)),
    ],
    out_shape=[
        jax.ShapeDtypeStruct((2 * NT, TILE), jnp.float32),
        jax.ShapeDtypeStruct((2 * NT, TILE), jnp.float32),
    ],
)


def _to_view3(a):
    return jnp.transpose(a.reshape(N // TILE, TILE, 2), (0, 2, 1))


def _to_view2(a):
    return _to_view3(a).reshape(2 * NT, TILE)


def _from_view3(f):
    return jnp.transpose(f, (0, 2, 1)).reshape(N, 2)


@jax.jit
def kernel(t, pos, vel, diameter):
    del t
    ddiam = _sc_call()
    dpos_f, dvel_f = _tc_call(
        _to_view2(pos), _to_view2(vel), diameter.reshape(NT, TILE))
    return (_from_view3(dpos_f.reshape(NT, 2, TILE)),
            _from_view3(dvel_f.reshape(NT, 2, TILE)),
            ddiam.reshape(N))


# speedup vs baseline: 1.0271x; 1.0271x over previous
import functools

import jax
import jax.numpy as jnp
from jax import lax
from jax.experimental import pallas as pl
from jax.experimental.pallas import tpu as pltpu
from jax.experimental.pallas import tpu_sc as plsc

N = 1048576
TILE = 128
NT = N // TILE
NUM_CORES = 2
NUM_SUBCORES = 16
NW = NUM_CORES * NUM_SUBCORES
T_PER_W = NT // NW
T_CHUNK = 64
SC_CHUNKS = T_PER_W // T_CHUNK
LANES = 16

BT = 256
GRID = NT // BT


def _sc_body(ddiam_hbm, zero_v):
    wid = lax.axis_index("s") * NUM_CORES + lax.axis_index("c")
    zeros = jnp.zeros((LANES,), jnp.float32)

    @plsc.parallel_loop(0, T_CHUNK * (TILE // LANES), unroll=8)
    def zbody(g):
        zero_v[g // 8, pl.ds((g % 8) * LANES, LANES)] = zeros

    for c in range(SC_CHUNKS):
        t0 = wid * T_PER_W + c * T_CHUNK
        pltpu.sync_copy(zero_v, ddiam_hbm.at[pl.ds(t0, T_CHUNK)])


_sc_call = functools.partial(
    pl.kernel,
    out_type=jax.ShapeDtypeStruct((NT, TILE), jnp.float32),
    mesh=plsc.VectorSubcoreMesh(core_axis_name="c", subcore_axis_name="s"),
    compiler_params=pltpu.CompilerParams(
        needs_layout_passes=False, use_tc_tiling_on_sc=False),
    scratch_types=[pltpu.VMEM((T_CHUNK, TILE), jnp.float32)],
)(_sc_body)


def _tc_body(pos_ref, vel_ref, diam_ref, dpos_ref, dvel_ref):
    y = pos_ref[:, 1, :]
    m = y <= 0.5 * diam_ref[...]
    dpos_ref[:, 0, :] = jnp.where(m, 0.0, vel_ref[:, 0, :])
    dpos_ref[:, 1, :] = jnp.where(m, 0.0, vel_ref[:, 1, :])
    dvel_ref[:, 0, :] = jnp.zeros((BT, TILE), jnp.float32)
    dvel_ref[:, 1, :] = jnp.where(m, 0.0, -20.0)


_tc_call = pl.pallas_call(
    _tc_body,
    grid=(GRID,),
    in_specs=[
        pl.BlockSpec((BT, 2, TILE), lambda i: (i, 0, 0)),
        pl.BlockSpec((BT, 2, TILE), lambda i: (i, 0, 0)),
        pl.BlockSpec((BT, TILE), lambda i: (i, 0)),
    ],
    out_specs=[
        pl.BlockSpec((BT, 2, TILE), lambda i: (i, 0, 0)),
        pl.BlockSpec((BT, 2, TILE), lambda i: (i, 0, 0)),
    ],
    out_shape=[
        jax.ShapeDtypeStruct((NT, 2, TILE), jnp.float32),
        jax.ShapeDtypeStruct((NT, 2, TILE), jnp.float32),
    ],
)


def _to_view3(a):
    return jnp.transpose(a.reshape(N // TILE, TILE, 2), (0, 2, 1))


def _to_view2(a):
    return _to_view3(a).reshape(2 * NT, TILE)


def _from_view3(f):
    return jnp.transpose(f, (0, 2, 1)).reshape(N, 2)


@jax.jit
def kernel(t, pos, vel, diameter):
    del t
    ddiam = _sc_call()
    dpos3, dvel3 = _tc_call(
        _to_view3(pos), _to_view3(vel), diameter.reshape(NT, TILE))
    return (_from_view3(dpos3), _from_view3(dvel3), ddiam.reshape(N))


# speedup vs baseline: 1.1240x; 1.0943x over previous
import functools

import jax
import jax.numpy as jnp
from jax import lax
from jax.experimental import pallas as pl
from jax.experimental.pallas import tpu as pltpu
from jax.experimental.pallas import tpu_sc as plsc

N = 1048576
TILE = 128
NT = N // TILE
NUM_CORES = 2
NUM_SUBCORES = 16
NW = NUM_CORES * NUM_SUBCORES
T_PER_W = NT // NW
T_CHUNK = 64
SC_CHUNKS = T_PER_W // T_CHUNK
LANES = 16
G_CHUNK = (T_CHUNK * TILE) // LANES

BT = 256
GRID = NT // BT


def _sc_body(pos_hbm, vel_hbm, diam_hbm, dpos_hbm, ddiam_hbm,
             y_v, vel_v, diam_v, zero_v, sem_in, sem_out):
    wid = lax.axis_index("s") * NUM_CORES + lax.axis_index("c")

    zeros = jnp.zeros((LANES,), jnp.float32)
    halves = jnp.full((LANES,), 0.5, jnp.float32)

    @plsc.parallel_loop(0, T_CHUNK * (TILE // LANES), unroll=8)
    def zbody(g):
        zero_v[g // 8, pl.ds((g % 8) * LANES, LANES)] = zeros

    def start_in(c, b):
        t0 = wid * T_PER_W + c * T_CHUNK
        return (
            pltpu.async_copy(
                pos_hbm.at[pl.ds(t0, T_CHUNK), 1], y_v.at[b], sem_in.at[b]),
            pltpu.async_copy(
                vel_hbm.at[pl.ds(2 * t0, 2 * T_CHUNK)], vel_v.at[b],
                sem_in.at[b]),
            pltpu.async_copy(
                diam_hbm.at[pl.ds(t0, T_CHUNK)], diam_v.at[b], sem_in.at[b]),
        )

    def start_out(c, b):
        t0 = wid * T_PER_W + c * T_CHUNK
        return (
            pltpu.async_copy(
                vel_v.at[b], dpos_hbm.at[pl.ds(2 * t0, 2 * T_CHUNK)],
                sem_out.at[b]),
            pltpu.async_copy(
                zero_v, ddiam_hbm.at[pl.ds(t0, T_CHUNK)], sem_out.at[b]),
        )

    in_flight = {0: start_in(0, 0)}
    out_flight = {}
    for c in range(SC_CHUNKS):
        b = c % 2
        for h in in_flight.pop(c):
            h.wait()

        y_b, vel_b, diam_b = y_v.at[b], vel_v.at[b], diam_v.at[b]

        @plsc.parallel_loop(0, G_CHUNK, unroll=8)
        def body(g):
            t = g // 8
            s = (g % 8) * LANES
            y = y_b[t, pl.ds(s, LANES)]
            d = diam_b[t, pl.ds(s, LANES)]
            m = y <= halves * d
            vx = vel_b[2 * t, pl.ds(s, LANES)]
            vy = vel_b[2 * t + 1, pl.ds(s, LANES)]
            vel_b[2 * t, pl.ds(s, LANES)] = jnp.where(m, zeros, vx)
            vel_b[2 * t + 1, pl.ds(s, LANES)] = jnp.where(m, zeros, vy)

        out_flight[c] = start_out(c, b)
        if c + 1 < SC_CHUNKS:
            nb = (c + 1) % 2
            if c - 1 >= 0:
                for h in out_flight.pop(c - 1):
                    h.wait()
            in_flight[c + 1] = start_in(c + 1, nb)
    for c in list(out_flight):
        for h in out_flight.pop(c):
            h.wait()


_sc_call = functools.partial(
    pl.kernel,
    out_type=(
        jax.ShapeDtypeStruct((2 * NT, TILE), jnp.float32),
        jax.ShapeDtypeStruct((NT, TILE), jnp.float32),
    ),
    mesh=plsc.VectorSubcoreMesh(core_axis_name="c", subcore_axis_name="s"),
    compiler_params=pltpu.CompilerParams(
        needs_layout_passes=False, use_tc_tiling_on_sc=False),
    scratch_types=[
        pltpu.VMEM((2, T_CHUNK, TILE), jnp.float32),
        pltpu.VMEM((2, 2 * T_CHUNK, TILE), jnp.float32),
        pltpu.VMEM((2, T_CHUNK, TILE), jnp.float32),
        pltpu.VMEM((T_CHUNK, TILE), jnp.float32),
        pltpu.SemaphoreType.DMA((2,)),
        pltpu.SemaphoreType.DMA((2,)),
    ],
)(_sc_body)


def _tc_body(pos_ref, diam_ref, dvel_ref):
    y = pos_ref[:, 1, :]
    m = y <= 0.5 * diam_ref[...]
    dvel_ref[:, 0, :] = jnp.zeros((BT, TILE), jnp.float32)
    dvel_ref[:, 1, :] = jnp.where(m, 0.0, -20.0)


_tc_call = pl.pallas_call(
    _tc_body,
    grid=(GRID,),
    in_specs=[
        pl.BlockSpec((BT, 2, TILE), lambda i: (i, 0, 0)),
        pl.BlockSpec((BT, TILE), lambda i: (i, 0)),
    ],
    out_specs=pl.BlockSpec((BT, 2, TILE), lambda i: (i, 0, 0)),
    out_shape=jax.ShapeDtypeStruct((NT, 2, TILE), jnp.float32),
)


def _to_view3(a):
    return jnp.transpose(a.reshape(N // TILE, TILE, 2), (0, 2, 1))


def _to_view2(a):
    return _to_view3(a).reshape(2 * NT, TILE)


def _from_view3(f):
    return jnp.transpose(f, (0, 2, 1)).reshape(N, 2)


@jax.jit
def kernel(t, pos, vel, diameter):
    del t
    diam2 = diameter.reshape(NT, TILE)
    dpos_f, ddiam = _sc_call(_to_view3(pos), _to_view2(vel), diam2)
    dvel3 = _tc_call(_to_view3(pos), diam2)
    return (_from_view3(dpos_f.reshape(NT, 2, TILE)), _from_view3(dvel3),
            ddiam.reshape(N))


# speedup vs baseline: 1.2847x; 1.1430x over previous
import functools

import jax
import jax.numpy as jnp
from jax import lax
from jax.experimental import pallas as pl
from jax.experimental.pallas import tpu as pltpu
from jax.experimental.pallas import tpu_sc as plsc

N = 1048576
TILE = 128
NT = N // TILE
NUM_CORES = 2
NUM_SUBCORES = 16
NW = NUM_CORES * NUM_SUBCORES
T_PER_W = NT // NW
T_CHUNK = 64
SC_CHUNKS = T_PER_W // T_CHUNK
LANES = 16
G_CHUNK = (T_CHUNK * TILE) // LANES

BT = 256
GRID = NT // BT


def _sc_body(pos_hbm, vel_hbm, diam_hbm, dpos_hbm, dvel_hbm,
             y_v, vel_v, diam_v, zero_v, sem_in, sem_out):
    wid = lax.axis_index("s") * NUM_CORES + lax.axis_index("c")

    zeros = jnp.zeros((LANES,), jnp.float32)
    neg20 = jnp.full((LANES,), -20.0, jnp.float32)
    halves = jnp.full((LANES,), 0.5, jnp.float32)

    @plsc.parallel_loop(0, T_CHUNK * (TILE // LANES), unroll=8)
    def zbody(g):
        zero_v[g // 8, pl.ds((g % 8) * LANES, LANES)] = zeros

    def start_in(c, b):
        t0 = wid * T_PER_W + c * T_CHUNK
        return (
            pltpu.async_copy(
                pos_hbm.at[pl.ds(t0, T_CHUNK), 1], y_v.at[b], sem_in.at[b]),
            pltpu.async_copy(
                vel_hbm.at[pl.ds(2 * t0, 2 * T_CHUNK)], vel_v.at[b],
                sem_in.at[b]),
            pltpu.async_copy(
                diam_hbm.at[pl.ds(t0, T_CHUNK)], diam_v.at[b], sem_in.at[b]),
        )

    def start_out(c, b):
        t0 = wid * T_PER_W + c * T_CHUNK
        return (
            pltpu.async_copy(
                vel_v.at[b], dpos_hbm.at[pl.ds(2 * t0, 2 * T_CHUNK)],
                sem_out.at[b]),
            pltpu.async_copy(
                y_v.at[b], dvel_hbm.at[pl.ds(t0, T_CHUNK), 1], sem_out.at[b]),
            pltpu.async_copy(
                zero_v, dvel_hbm.at[pl.ds(t0, T_CHUNK), 0], sem_out.at[b]),
        )

    in_flight = {0: start_in(0, 0)}
    out_flight = {}
    for c in range(SC_CHUNKS):
        b = c % 2
        for h in in_flight.pop(c):
            h.wait()

        y_b, vel_b, diam_b = y_v.at[b], vel_v.at[b], diam_v.at[b]

        @plsc.parallel_loop(0, G_CHUNK, unroll=8)
        def body(g):
            t = g // 8
            s = (g % 8) * LANES
            y = y_b[t, pl.ds(s, LANES)]
            d = diam_b[t, pl.ds(s, LANES)]
            m = y <= halves * d
            vx = vel_b[2 * t, pl.ds(s, LANES)]
            vy = vel_b[2 * t + 1, pl.ds(s, LANES)]
            vel_b[2 * t, pl.ds(s, LANES)] = jnp.where(m, zeros, vx)
            vel_b[2 * t + 1, pl.ds(s, LANES)] = jnp.where(m, zeros, vy)
            y_b[t, pl.ds(s, LANES)] = jnp.where(m, zeros, neg20)

        out_flight[c] = start_out(c, b)
        if c + 1 < SC_CHUNKS:
            nb = (c + 1) % 2
            if c - 1 >= 0:
                for h in out_flight.pop(c - 1):
                    h.wait()
            in_flight[c + 1] = start_in(c + 1, nb)
    for c in list(out_flight):
        for h in out_flight.pop(c):
            h.wait()


_sc_call = functools.partial(
    pl.kernel,
    out_type=(
        jax.ShapeDtypeStruct((2 * NT, TILE), jnp.float32),
        jax.ShapeDtypeStruct((NT, 2, TILE), jnp.float32),
    ),
    mesh=plsc.VectorSubcoreMesh(core_axis_name="c", subcore_axis_name="s"),
    compiler_params=pltpu.CompilerParams(
        needs_layout_passes=False, use_tc_tiling_on_sc=False),
    scratch_types=[
        pltpu.VMEM((2, T_CHUNK, TILE), jnp.float32),
        pltpu.VMEM((2, 2 * T_CHUNK, TILE), jnp.float32),
        pltpu.VMEM((2, T_CHUNK, TILE), jnp.float32),
        pltpu.VMEM((T_CHUNK, TILE), jnp.float32),
        pltpu.SemaphoreType.DMA((2,)),
        pltpu.SemaphoreType.DMA((2,)),
    ],
)(_sc_body)


def _tc_body(ddiam_ref):
    ddiam_ref[...] = jnp.zeros((BT, TILE), jnp.float32)


_tc_call = pl.pallas_call(
    _tc_body,
    grid=(GRID,),
    out_specs=pl.BlockSpec((BT, TILE), lambda i: (i, 0)),
    out_shape=jax.ShapeDtypeStruct((NT, TILE), jnp.float32),
)


def _to_view3(a):
    return jnp.transpose(a.reshape(N // TILE, TILE, 2), (0, 2, 1))


def _to_view2(a):
    return _to_view3(a).reshape(2 * NT, TILE)


def _from_view3(f):
    return jnp.transpose(f, (0, 2, 1)).reshape(N, 2)


@jax.jit
def kernel(t, pos, vel, diameter):
    del t
    diam2 = diameter.reshape(NT, TILE)
    dpos_f, dvel3 = _sc_call(_to_view3(pos), _to_view2(vel), diam2)
    ddiam = _tc_call()
    return (_from_view3(dpos_f.reshape(NT, 2, TILE)), _from_view3(dvel3),
            ddiam.reshape(N))
